# baseline (speedup 1.0000x reference)
import jax
import jax.numpy as jnp
from jax.experimental import pallas as pl


def _prep_body(t_ref, trep_ref, ts_ref, esk_ref, oh_ref, m_ref):
    m = ts_ref.shape[1]
    t = t_ref[:, :]
    ts = ts_ref[:, :]
    lo = ts[0, 0]
    hi = ts[0, m - 1]
    tc = jnp.clip(t, lo, hi)
    idx = jnp.sum((ts < tc).astype(jnp.int32), axis=1, keepdims=True)
    idx = jnp.clip(idx, 1, m - 1)
    c16 = jax.lax.broadcasted_iota(jnp.int32, (t.shape[0], 2 * m), 1)
    oh_ref[:, :] = ((c16 == idx - 1) | (c16 == idx + m)).astype(jnp.float32)

    tr = trep_ref[:, :]
    trc = jnp.clip(tr, lo, hi)
    idxc = jnp.zeros(tr.shape, jnp.int32)
    for mm in range(m):
        idxc += (ts[0, mm] < trc).astype(jnp.int32)
    idxc = jnp.clip(idxc, 1, m - 1)
    t0 = jnp.zeros(tr.shape, jnp.float32)
    t1 = jnp.zeros(tr.shape, jnp.float32)
    for mm in range(m):
        t0 = jnp.where(idxc - 1 == mm, ts[0, mm], t0)
        t1 = jnp.where(idxc == mm, ts[0, mm], t1)
    w = (trc - t0) / (t1 - t0 + 1e-12)
    esk = esk_ref[:, :]
    m_ref[0:m, :] = esk * (1.0 - w)
    m_ref[m:2 * m, :] = esk * w


def _matmul_body(oh_ref, m_ref, o_ref):
    o_ref[:, :] = jnp.dot(oh_ref[:, :], m_ref[:, :],
                          preferred_element_type=jnp.float32)


def kernel(t, ts, Es):
    B = t.shape[0]
    m = ts.shape[0]
    k = Es.shape[1]
    C = B * k

    ts2 = ts.reshape(1, m)
    trep = jnp.broadcast_to(t, (B, k)).reshape(1, C)
    esk = jnp.tile(Es, (1, B))

    OH, M = pl.pallas_call(
        _prep_body,
        out_shape=(
            jax.ShapeDtypeStruct((B, 2 * m), jnp.float32),
            jax.ShapeDtypeStruct((2 * m, C), jnp.float32),
        ),
    )(t, trep, ts2, esk)

    BI = 256
    out = pl.pallas_call(
        _matmul_body,
        grid=(B // BI,),
        in_specs=[
            pl.BlockSpec((BI, 2 * m), lambda i: (i, 0)),
            pl.BlockSpec((2 * m, C), lambda i: (0, 0)),
        ],
        out_specs=pl.BlockSpec((BI, C), lambda i: (i, 0)),
        out_shape=jax.ShapeDtypeStruct((B, C), jnp.float32),
    )(OH, M)

    return out.reshape(B, B, k)


# speedup vs baseline: 4.6173x; 4.6173x over previous
import jax
import jax.numpy as jnp
from jax.experimental import pallas as pl


def _prep_body(t_ref, trow_ref, ts_ref, es_ref, w_ref, a_ref, d_ref):
    m = ts_ref.shape[1]
    ts = ts_ref[:, :]
    lo = ts[0, 0]
    hi = ts[0, m - 1]

    t = t_ref[:, :]
    tc = jnp.clip(t, lo, hi)
    idx = jnp.sum((ts < tc).astype(jnp.int32), axis=1, keepdims=True)
    idx = jnp.clip(idx, 1, m - 1)
    cols = jax.lax.broadcasted_iota(jnp.int32, (t.shape[0], m), 1)
    oh0 = (cols == idx - 1).astype(jnp.float32)
    oh1 = (cols == idx).astype(jnp.float32)
    es = es_ref[:, :]
    e0 = jnp.dot(oh0, es, preferred_element_type=jnp.float32)
    e1 = jnp.dot(oh1, es, preferred_element_type=jnp.float32)
    a_ref[:, :] = e0
    d_ref[:, :] = e1 - e0

    tr = trow_ref[:, :]
    trc = jnp.clip(tr, lo, hi)
    idxc = jnp.zeros(tr.shape, jnp.int32)
    for mm in range(m):
        idxc += (ts[0, mm] < trc).astype(jnp.int32)
    idxc = jnp.clip(idxc, 1, m - 1)
    t0 = jnp.zeros(tr.shape, jnp.float32)
    t1 = jnp.zeros(tr.shape, jnp.float32)
    for mm in range(m):
        t0 = jnp.where(idxc - 1 == mm, ts[0, mm], t0)
        t1 = jnp.where(idxc == mm, ts[0, mm], t1)
    w_ref[:, :] = (trc - t0) / (t1 - t0 + 1e-12)


def _bcast_body(a_ref, d_ref, w_ref, o_ref):
    a = a_ref[:, :]
    d = d_ref[:, :]
    w = w_ref[:, :]
    o_ref[:, :] = a + d * w


def kernel(t, ts, Es):
    B = t.shape[0]
    m = ts.shape[0]
    k = Es.shape[1]
    R = B * k

    ts2 = ts.reshape(1, m)
    trow = t.reshape(1, B)

    w, A, D = pl.pallas_call(
        _prep_body,
        out_shape=(
            jax.ShapeDtypeStruct((1, B), jnp.float32),
            jax.ShapeDtypeStruct((B, k), jnp.float32),
            jax.ShapeDtypeStruct((B, k), jnp.float32),
        ),
    )(t, trow, ts2, Es)

    a = A.reshape(R, 1)
    d = D.reshape(R, 1)

    BR = 1024
    q = pl.pallas_call(
        _bcast_body,
        grid=(R // BR,),
        in_specs=[
            pl.BlockSpec((BR, 1), lambda i: (i, 0)),
            pl.BlockSpec((BR, 1), lambda i: (i, 0)),
            pl.BlockSpec((1, B), lambda i: (0, 0)),
        ],
        out_specs=pl.BlockSpec((BR, B), lambda i: (i, 0)),
        out_shape=jax.ShapeDtypeStruct((R, B), jnp.float32),
    )(a, d, w)

    return q.reshape(B, k, B).transpose(0, 2, 1)


# speedup vs baseline: 4.6518x; 1.0075x over previous
import jax
import jax.numpy as jnp
from jax.experimental import pallas as pl


def _prep_body(t_ref, trow_ref, ts_ref, es_ref, w_ref, a_ref, d_ref):
    m = ts_ref.shape[1]
    ts = ts_ref[:, :]
    lo = ts[0, 0]
    hi = ts[0, m - 1]

    t = t_ref[:, :]
    tc = jnp.clip(t, lo, hi)
    idx = jnp.sum((ts < tc).astype(jnp.int32), axis=1, keepdims=True)
    idx = jnp.clip(idx, 1, m - 1)
    cols = jax.lax.broadcasted_iota(jnp.int32, (t.shape[0], m), 1)
    oh0 = (cols == idx - 1).astype(jnp.float32)
    oh1 = (cols == idx).astype(jnp.float32)
    es = es_ref[:, :]
    e0 = jnp.dot(oh0, es, preferred_element_type=jnp.float32)
    e1 = jnp.dot(oh1, es, preferred_element_type=jnp.float32)
    a_ref[:, :] = e0
    d_ref[:, :] = e1 - e0

    tr = trow_ref[:, :]
    trc = jnp.clip(tr, lo, hi)
    idxc = jnp.zeros(tr.shape, jnp.int32)
    for mm in range(m):
        idxc += (ts[0, mm] < trc).astype(jnp.int32)
    idxc = jnp.clip(idxc, 1, m - 1)
    t0 = jnp.zeros(tr.shape, jnp.float32)
    t1 = jnp.zeros(tr.shape, jnp.float32)
    for mm in range(m):
        t0 = jnp.where(idxc - 1 == mm, ts[0, mm], t0)
        t1 = jnp.where(idxc == mm, ts[0, mm], t1)
    w_ref[:, :] = (trc - t0) / (t1 - t0 + 1e-12)


def _bcast_body(a_ref, d_ref, w_ref, o_ref):
    a = a_ref[:, :]
    d = d_ref[:, :]
    w = w_ref[:, :]
    o_ref[:, :] = a + d * w


def kernel(t, ts, Es):
    B = t.shape[0]
    m = ts.shape[0]
    k = Es.shape[1]
    R = B * k

    ts2 = ts.reshape(1, m)
    trow = t.reshape(1, B)

    w, A, D = pl.pallas_call(
        _prep_body,
        out_shape=(
            jax.ShapeDtypeStruct((1, B), jnp.float32),
            jax.ShapeDtypeStruct((B, k), jnp.float32),
            jax.ShapeDtypeStruct((B, k), jnp.float32),
        ),
    )(t, trow, ts2, Es)

    a = A.reshape(R, 1)
    d = D.reshape(R, 1)

    BR = 2048
    q = pl.pallas_call(
        _bcast_body,
        grid=(R // BR,),
        in_specs=[
            pl.BlockSpec((BR, 1), lambda i: (i, 0)),
            pl.BlockSpec((BR, 1), lambda i: (i, 0)),
            pl.BlockSpec((1, B), lambda i: (0, 0)),
        ],
        out_specs=pl.BlockSpec((BR, B), lambda i: (i, 0)),
        out_shape=jax.ShapeDtypeStruct((R, B), jnp.float32),
    )(a, d, w)

    return q.reshape(B, k, B).transpose(0, 2, 1)


# speedup vs baseline: 4.8599x; 1.0447x over previous
import jax
import jax.numpy as jnp
from jax.experimental import pallas as pl


def _prep_body(trow_ref, ts_ref, es_ref, w_ref, a_ref, d_ref):
    m = ts_ref.shape[1]
    B = trow_ref.shape[1]
    ts = ts_ref[:, :]
    lo = ts[0, 0]
    hi = ts[0, m - 1]

    tr = trow_ref[:, :]
    trc = jnp.clip(tr, lo, hi)
    idxc = jnp.zeros(tr.shape, jnp.int32)
    for mm in range(m):
        idxc += (ts[0, mm] < trc).astype(jnp.int32)
    idxc = jnp.clip(idxc, 1, m - 1)
    t0 = jnp.zeros(tr.shape, jnp.float32)
    t1 = jnp.zeros(tr.shape, jnp.float32)
    for mm in range(m):
        t0 = jnp.where(idxc - 1 == mm, ts[0, mm], t0)
        t1 = jnp.where(idxc == mm, ts[0, mm], t1)
    w_ref[:, :] = (trc - t0) / (t1 - t0 + 1e-12)

    rows = jax.lax.broadcasted_iota(jnp.int32, (m, B), 0)
    p0 = (rows == (idxc - 1)).astype(jnp.float32)
    p1 = (rows == idxc).astype(jnp.float32)
    es = es_ref[:, :]
    dn = (((0,), (0,)), ((), ()))
    e0 = jax.lax.dot_general(p0, es, dn,
                             preferred_element_type=jnp.float32)
    e1 = jax.lax.dot_general(p1, es, dn,
                             preferred_element_type=jnp.float32)
    a_ref[:, :] = e0
    d_ref[:, :] = e1 - e0


def _bcast_body(a_ref, d_ref, w_ref, o_ref):
    a = a_ref[:, :]
    d = d_ref[:, :]
    w = w_ref[:, :]
    o_ref[:, :] = a + d * w


def kernel(t, ts, Es):
    B = t.shape[0]
    m = ts.shape[0]
    k = Es.shape[1]
    R = B * k

    ts2 = ts.reshape(1, m)
    trow = t.reshape(1, B)

    w, A, D = pl.pallas_call(
        _prep_body,
        out_shape=(
            jax.ShapeDtypeStruct((1, B), jnp.float32),
            jax.ShapeDtypeStruct((B, k), jnp.float32),
            jax.ShapeDtypeStruct((B, k), jnp.float32),
        ),
    )(trow, ts2, Es)

    a = A.reshape(R, 1)
    d = D.reshape(R, 1)

    BR = 2048
    q = pl.pallas_call(
        _bcast_body,
        grid=(R // BR,),
        in_specs=[
            pl.BlockSpec((BR, 1), lambda i: (i, 0)),
            pl.BlockSpec((BR, 1), lambda i: (i, 0)),
            pl.BlockSpec((1, B), lambda i: (0, 0)),
        ],
        out_specs=pl.BlockSpec((BR, B), lambda i: (i, 0)),
        out_shape=jax.ShapeDtypeStruct((R, B), jnp.float32),
    )(a, d, w)

    return q.reshape(B, k, B).transpose(0, 2, 1)
